# baseline (speedup 1.0000x reference)
import functools

import jax
import jax.numpy as jnp
from jax import lax
from jax.experimental import pallas as pl
from jax.experimental.pallas import tpu as pltpu
from jax.experimental.pallas import tpu_sc as plsc

T = 16
B, S, D = 2, 2048, 1024
NW = 32
R = 2
CW = R * D
ROWS_PER_W = S // NW
CHUNKS = ROWS_PER_W // R
VPC = CW // 16
NSEG = CW // 128
PR = S * D // 128
QB = 256


def _sc_body(x_hbm, out_hbm, xbuf0, xbuf1, ob0, ob1, st0, st1,
             isem0, isem1, osem0, osem1):
    wid = lax.axis_index("s") * 2 + lax.axis_index("c")
    row0 = wid * ROWS_PER_W

    iota = lax.iota(jnp.int32, 16)
    ones = jnp.full((16,), 1.0, jnp.float32)
    zeros = jnp.zeros((16,), jnp.float32)
    izeros = jnp.zeros((16,), jnp.int32)

    xbufs = (xbuf0, xbuf1)
    obufs = (ob0, ob1)
    stbufs = (st0, st1)
    isems = (isem0, isem1)
    osems = (osem0, osem1)

    @plsc.parallel_loop(0, T * CW // 16, unroll=4)
    def _zero(i):
        ob0[i >> 7, (i >> 3) & 15, pl.ds((i & 7) * 16, 16)] = zeros
        ob1[i >> 7, (i >> 3) & 15, pl.ds((i & 7) * 16, 16)] = zeros

    @plsc.parallel_loop(0, VPC, unroll=4)
    def _zero_st(i):
        st0[pl.ds(i * 16, 16)] = izeros
        st1[pl.ds(i * 16, 16)] = izeros

    for slot in range(2):
        pltpu.async_copy(
            x_hbm.at[pl.ds((row0 + slot * R) * D, CW)], xbufs[slot], isems[slot]
        )

    def outer(c2, _):
        for slot in range(2):
            xbuf, obuf, stbuf = xbufs[slot], obufs[slot], stbufs[slot]
            isem, osem = isems[slot], osems[slot]
            c = c2 * 2 + slot
            s0 = row0 + c * R

            pltpu.make_async_copy(x_hbm.at[pl.ds(0, CW)], xbuf, isem).wait()

            @pl.when(c2 >= 1)
            def _drain_out():
                pltpu.make_async_copy(
                    out_hbm.at[pl.ds(0, T), pl.ds(0, NSEG), :], obuf, osem
                ).wait()

            @plsc.parallel_loop(0, VPC, unroll=8)
            def _encode(i):
                rowv = jnp.broadcast_to((i >> 3) & 15, (16,)).astype(jnp.int32)
                lanev = (i & 7) * 16 + iota
                xv = xbuf[pl.ds(i * 16, 16)]
                e = jnp.exp(-jnp.abs(xv))
                sig = jnp.where(xv >= 0.0, 1.0, e) / (1.0 + e)
                stv = (sig * 15.0).astype(jnp.int32)
                old = stbuf[pl.ds(i * 16, 16)]
                clear = jnp.where(old == stv, 1.0, 0.0)
                plsc.store_scatter(obuf, [old, rowv, lanev], clear)
                plsc.store_scatter(obuf, [stv, rowv, lanev], ones)
                stbuf[pl.ds(i * 16, 16)] = stv

            @pl.when(c2 < CHUNKS // 2 - 1)
            def _prefetch():
                pltpu.async_copy(
                    x_hbm.at[pl.ds((s0 + 2 * R) * D, CW)], xbuf, isem
                )

            pltpu.async_copy(
                obuf,
                out_hbm.at[pl.ds(0, T), pl.ds(s0 * (D // 128), NSEG), :],
                osem,
            )
        return 0

    lax.fori_loop(0, CHUNKS // 2, outer, 0)

    for slot in range(2):
        pltpu.make_async_copy(
            out_hbm.at[pl.ds(0, T), pl.ds(0, NSEG), :], obufs[slot], osems[slot]
        ).wait()


def _sc_encode(xf0):
    k = functools.partial(
        pl.kernel,
        out_type=jax.ShapeDtypeStruct((T, PR, 128), jnp.float32),
        mesh=plsc.VectorSubcoreMesh(core_axis_name="c", subcore_axis_name="s"),
        compiler_params=pltpu.CompilerParams(needs_layout_passes=False),
        scratch_types=[
            pltpu.VMEM((CW,), jnp.float32),
            pltpu.VMEM((CW,), jnp.float32),
            pltpu.VMEM((T, NSEG, 128), jnp.float32),
            pltpu.VMEM((T, NSEG, 128), jnp.float32),
            pltpu.VMEM((CW,), jnp.int32),
            pltpu.VMEM((CW,), jnp.int32),
            pltpu.SemaphoreType.DMA,
            pltpu.SemaphoreType.DMA,
            pltpu.SemaphoreType.DMA,
            pltpu.SemaphoreType.DMA,
        ],
    )(_sc_body)
    return k(xf0)


def _tc_body(x_ref, o_ref):
    x = x_ref[...]
    st = (jax.nn.sigmoid(x) * (T - 1)).astype(jnp.int32)
    t_iota = jax.lax.broadcasted_iota(jnp.int32, (T,) + st.shape, 0)
    o_ref[...] = (st[None] == t_iota).astype(jnp.float32)


def _tc_onehot(x1):
    return pl.pallas_call(
        _tc_body,
        grid=(PR // QB,),
        in_specs=[pl.BlockSpec((QB, 128), lambda q: (q, 0))],
        out_specs=pl.BlockSpec((T, QB, 128), lambda q: (0, q, 0)),
        out_shape=jax.ShapeDtypeStruct((T, PR, 128), jnp.float32),
    )(x1)


@jax.jit
def _encode(x):
    xf = (
        x.reshape(B, S // 8, 8, D // 128, 128)
        .transpose(0, 1, 3, 2, 4)
        .reshape(B, S * D)
    )
    sc_out = _sc_encode(xf[0])
    tc_out = _tc_onehot(xf[1].reshape(PR, 128))
    out = jnp.concatenate([sc_out, tc_out], axis=0)
    return (
        out.reshape(B, T, S // 8, D // 128, 8, 128)
        .transpose(0, 1, 2, 4, 3, 5)
        .reshape(B, T, S, D)
    )


def kernel(x):
    return _encode(x)


# speedup vs baseline: 2.4787x; 2.4787x over previous
import functools

import jax
import jax.numpy as jnp
from jax import lax
from jax.experimental import pallas as pl
from jax.experimental.pallas import tpu as pltpu
from jax.experimental.pallas import tpu_sc as plsc

T = 16
B, S, D = 2, 2048, 1024
NW = 32
R = 2
CW = R * D
ROWS_PER_W = (B * S) // NW
CHUNKS = ROWS_PER_W // R
VPC = CW // 16
NSEG = CW // 128


def _sc_body(x_hbm, out_hbm, xbuf0, xbuf1, ob0, ob1, st0, st1,
             isem0, isem1, osem0, osem1):
    wid = lax.axis_index("s") * 2 + lax.axis_index("c")
    row0 = wid * ROWS_PER_W

    iota = lax.iota(jnp.int32, 16)
    ones = jnp.full((16,), 1.0, jnp.float32)
    zeros = jnp.zeros((16,), jnp.float32)
    izeros = jnp.zeros((16,), jnp.int32)

    xbufs = (xbuf0, xbuf1)
    obufs = (ob0, ob1)
    stbufs = (st0, st1)
    isems = (isem0, isem1)
    osems = (osem0, osem1)

    @plsc.parallel_loop(0, T * CW // 16, unroll=4)
    def _zero(i):
        ob0[i >> 7, (i >> 3) & 15, pl.ds((i & 7) * 16, 16)] = zeros
        ob1[i >> 7, (i >> 3) & 15, pl.ds((i & 7) * 16, 16)] = zeros

    @plsc.parallel_loop(0, VPC, unroll=4)
    def _zero_st(i):
        st0[pl.ds(i * 16, 16)] = izeros
        st1[pl.ds(i * 16, 16)] = izeros

    for slot in range(2):
        pltpu.async_copy(
            x_hbm.at[pl.ds((row0 + slot * R) * D, CW)], xbufs[slot], isems[slot]
        )

    def outer(c2, _):
        for slot in range(2):
            xbuf, obuf, stbuf = xbufs[slot], obufs[slot], stbufs[slot]
            isem, osem = isems[slot], osems[slot]
            c = c2 * 2 + slot
            n0 = row0 + c * R
            b = n0 >> 11
            s0 = n0 & 2047

            pltpu.make_async_copy(x_hbm.at[pl.ds(0, CW)], xbuf, isem).wait()

            @pl.when(c2 >= 1)
            def _drain_out():
                pltpu.make_async_copy(
                    out_hbm.at[pl.ds(0, T), pl.ds(0, NSEG), :], obuf, osem
                ).wait()

            @plsc.parallel_loop(0, VPC, unroll=16)
            def _encode(i):
                rowv = jnp.broadcast_to((i >> 3) & 15, (16,)).astype(jnp.int32)
                lanev = (i & 7) * 16 + iota
                xv = xbuf[pl.ds(i * 16, 16)]
                e = jnp.exp(-jnp.abs(xv))
                sig = jnp.where(xv >= 0.0, 1.0, e) / (1.0 + e)
                stv = (sig * 15.0).astype(jnp.int32)
                old = stbuf[pl.ds(i * 16, 16)]
                clear = jnp.where(old == stv, 1.0, 0.0)
                plsc.store_scatter(obuf, [old, rowv, lanev], clear)
                plsc.store_scatter(obuf, [stv, rowv, lanev], ones)
                stbuf[pl.ds(i * 16, 16)] = stv

            @pl.when(c2 < CHUNKS // 2 - 1)
            def _prefetch():
                pltpu.async_copy(
                    x_hbm.at[pl.ds((n0 + 2 * R) * D, CW)], xbuf, isem
                )

            pltpu.async_copy(
                obuf,
                out_hbm.at[pl.ds(b * T, T), pl.ds(s0 * (D // 128), NSEG), :],
                osem,
            )
        return 0

    lax.fori_loop(0, CHUNKS // 2, outer, 0)

    for slot in range(2):
        pltpu.make_async_copy(
            out_hbm.at[pl.ds(0, T), pl.ds(0, NSEG), :], obufs[slot], osems[slot]
        ).wait()


@jax.jit
def _sc_encode(xf):
    k = functools.partial(
        pl.kernel,
        out_type=jax.ShapeDtypeStruct((B * T, S * D // 128, 128), jnp.float32),
        mesh=plsc.VectorSubcoreMesh(core_axis_name="c", subcore_axis_name="s"),
        compiler_params=pltpu.CompilerParams(needs_layout_passes=False),
        scratch_types=[
            pltpu.VMEM((CW,), jnp.float32),
            pltpu.VMEM((CW,), jnp.float32),
            pltpu.VMEM((T, NSEG, 128), jnp.float32),
            pltpu.VMEM((T, NSEG, 128), jnp.float32),
            pltpu.VMEM((CW,), jnp.int32),
            pltpu.VMEM((CW,), jnp.int32),
            pltpu.SemaphoreType.DMA,
            pltpu.SemaphoreType.DMA,
            pltpu.SemaphoreType.DMA,
            pltpu.SemaphoreType.DMA,
        ],
    )(_sc_body)
    return k(xf)


def kernel(x):
    xf = (
        x.reshape(B, S // 8, 8, D // 128, 128)
        .transpose(0, 1, 3, 2, 4)
        .reshape(-1)
    )
    out = _sc_encode(xf)
    return (
        out.reshape(B, T, S // 8, D // 128, 8, 128)
        .transpose(0, 1, 2, 4, 3, 5)
        .reshape(B, T, S, D)
    )


# speedup vs baseline: 2.5003x; 1.0087x over previous
import functools

import jax
import jax.numpy as jnp
from jax import lax
from jax.experimental import pallas as pl
from jax.experimental.pallas import tpu as pltpu
from jax.experimental.pallas import tpu_sc as plsc

T = 16
B, S, D = 2, 2048, 1024
NW = 32
R = 2
CW = R * D
ROWS_PER_W = (B * S) // NW
CHUNKS = ROWS_PER_W // R
VPC = CW // 16
NSEG = CW // 128


def _sc_body(x_hbm, out_hbm, xbuf0, xbuf1, ob0, ob1, st0, st1,
             isem0, isem1, osem0, osem1):
    wid = lax.axis_index("s") * 2 + lax.axis_index("c")
    row0 = wid * ROWS_PER_W

    iota = lax.iota(jnp.int32, 16)
    ones = jnp.full((16,), 1.0, jnp.float32)
    zeros = jnp.zeros((16,), jnp.float32)
    izeros = jnp.zeros((16,), jnp.int32)

    xbufs = (xbuf0, xbuf1)
    obufs = (ob0, ob1)
    stbufs = (st0, st1)
    isems = (isem0, isem1)
    osems = (osem0, osem1)

    @plsc.parallel_loop(0, T * CW // 16, unroll=4)
    def _zero(i):
        ob0[i >> 7, (i >> 3) & 15, pl.ds((i & 7) * 16, 16)] = zeros
        ob1[i >> 7, (i >> 3) & 15, pl.ds((i & 7) * 16, 16)] = zeros

    @plsc.parallel_loop(0, VPC, unroll=4)
    def _zero_st(i):
        st0[pl.ds(i * 16, 16)] = izeros
        st1[pl.ds(i * 16, 16)] = izeros

    for slot in range(2):
        pltpu.async_copy(
            x_hbm.at[pl.ds((row0 + slot * R) * D, CW)], xbufs[slot], isems[slot]
        )

    def outer(c2, _):
        for slot in range(2):
            xbuf, obuf, stbuf = xbufs[slot], obufs[slot], stbufs[slot]
            isem, osem = isems[slot], osems[slot]
            c = c2 * 2 + slot
            n0 = row0 + c * R
            b = n0 >> 11
            s0 = n0 & 2047

            pltpu.make_async_copy(x_hbm.at[pl.ds(0, CW)], xbuf, isem).wait()

            @pl.when(c2 >= 1)
            def _drain_out():
                pltpu.make_async_copy(
                    out_hbm.at[pl.ds(0, T), pl.ds(0, NSEG), :], obuf, osem
                ).wait()

            @plsc.parallel_loop(0, VPC, unroll=8)
            def _encode(i):
                rowv = jnp.broadcast_to((i >> 3) & 15, (16,)).astype(jnp.int32)
                lanev = (i & 7) * 16 + iota
                xv = xbuf[pl.ds(i * 16, 16)]
                e = jnp.exp(-jnp.abs(xv))
                sig = jnp.where(xv >= 0.0, 1.0, e) / (1.0 + e)
                stv = (sig * 15.0).astype(jnp.int32)
                old = stbuf[pl.ds(i * 16, 16)]
                clear = jnp.where(old == stv, 1.0, 0.0)
                plsc.store_scatter(obuf, [old, rowv, lanev], clear)
                plsc.store_scatter(obuf, [stv, rowv, lanev], ones)
                stbuf[pl.ds(i * 16, 16)] = stv

            @pl.when(c2 < CHUNKS // 2 - 1)
            def _prefetch():
                pltpu.async_copy(
                    x_hbm.at[pl.ds((n0 + 2 * R) * D, CW)], xbuf, isem
                )

            pltpu.async_copy(
                obuf,
                out_hbm.at[pl.ds(b * T, T), pl.ds(s0 * (D // 128), NSEG), :],
                osem,
            )
        return 0

    lax.fori_loop(0, CHUNKS // 2, outer, 0)

    for slot in range(2):
        pltpu.make_async_copy(
            out_hbm.at[pl.ds(0, T), pl.ds(0, NSEG), :], obufs[slot], osems[slot]
        ).wait()


@jax.jit
def _sc_encode(xf):
    k = functools.partial(
        pl.kernel,
        out_type=jax.ShapeDtypeStruct((B * T, S * D // 128, 128), jnp.float32),
        mesh=plsc.VectorSubcoreMesh(core_axis_name="c", subcore_axis_name="s"),
        compiler_params=pltpu.CompilerParams(needs_layout_passes=False),
        scratch_types=[
            pltpu.VMEM((CW,), jnp.float32),
            pltpu.VMEM((CW,), jnp.float32),
            pltpu.VMEM((T, NSEG, 128), jnp.float32),
            pltpu.VMEM((T, NSEG, 128), jnp.float32),
            pltpu.VMEM((CW,), jnp.int32),
            pltpu.VMEM((CW,), jnp.int32),
            pltpu.SemaphoreType.DMA,
            pltpu.SemaphoreType.DMA,
            pltpu.SemaphoreType.DMA,
            pltpu.SemaphoreType.DMA,
        ],
    )(_sc_body)
    return k(xf)


def kernel(x):
    xf = (
        x.reshape(B, S // 8, 8, D // 128, 128)
        .transpose(0, 1, 3, 2, 4)
        .reshape(-1)
    )
    out = _sc_encode(xf)
    return (
        out.reshape(B, T, S // 8, D // 128, 8, 128)
        .transpose(0, 1, 2, 4, 3, 5)
        .reshape(B, T, S, D)
    )
